# baseline (speedup 1.0000x reference)
import jax
import jax.numpy as jnp
from jax import lax
from jax.experimental import pallas as pl
from jax.experimental.pallas import tpu as pltpu
from jax.experimental.pallas import tpu_sc as plsc

NS = 16
L = 16
CHUNK = 128
GRP = 4
F = 16


def _sc_mesh():
  return plsc.VectorSubcoreMesh(
      core_axis_name="c", subcore_axis_name="s", num_cores=1,
      num_subcores=NS)


def _rsqrt16(x):
  xi = plsc.bitcast(x, jnp.int32)
  gi = jnp.int32(0x5F3759DF) - lax.shift_right_logical(xi, 1)
  g = plsc.bitcast(gi, jnp.float32)
  half = -0.5 * x
  for _ in range(3):
    g = g * (1.5 + half * g * g)
  return jnp.where(x > 0.0, g, 0.0)


def _agg_phase(srcs, col_v, row_v, bufs, acc, gsems, ssems, n_chunks):
  n_groups = n_chunks // GRP

  def gather(i):
    p = i % 2
    for b in range(GRP):
      j = i * GRP + b
      pltpu.async_copy(srcs[p].at[col_v.at[j]], bufs.at[p, b], gsems[p])

  gather(0)
  for i in range(n_groups):
    p = i % 2
    if i + 1 < n_groups:
      if i >= 1:
        for b in range(GRP):
          pltpu.make_async_copy(bufs.at[1 - p, b],
                                acc.at[row_v.at[b]], ssems[1 - p]).wait()
      gather(i + 1)
    for b in range(GRP):
      j = i * GRP + b
      pltpu.make_async_copy(srcs[p].at[col_v.at[j]], bufs.at[p, b],
                            gsems[p]).wait()
    for b in range(GRP):
      j = i * GRP + b
      pltpu.async_copy(bufs.at[p, b], acc.at[row_v.at[j]], ssems[p],
                       add=True)
  for i in (n_groups - 2, n_groups - 1):
    p = i % 2
    for b in range(GRP):
      pltpu.make_async_copy(bufs.at[p, b], acc.at[row_v.at[b]],
                            ssems[p]).wait()


_AGG_SCRATCH = lambda n_pad, n_chunks: [
    pltpu.VMEM((n_chunks, CHUNK), jnp.int32),
    pltpu.VMEM((n_chunks, CHUNK), jnp.int32),
    pltpu.VMEM((2, GRP, CHUNK, F), jnp.float32),
    pltpu.VMEM_SHARED((n_pad, F), jnp.float32),
    [pltpu.SemaphoreType.DMA, pltpu.SemaphoreType.DMA],
    [pltpu.SemaphoreType.DMA, pltpu.SemaphoreType.DMA],
]


def _make_sc_layer1(n_pad, n_chunks):
  rows_per_tile = n_pad // NS
  nvec = rows_per_tile // L

  def body(cols_hbm, rows_hbm, xw1_hbm, acc_out, dis_out, ytab_hbm,
           col_v, row_v, bufs, acc, gsems, ssems,
           hist, sumbuf, comb, xw_v, y_v, dis_v, csem, ytab):
    s = lax.axis_index("s")

    pltpu.sync_copy(cols_hbm.at[s], col_v)
    pltpu.sync_copy(rows_hbm.at[s], row_v)

    zvec = jnp.zeros((L,), jnp.float32)
    ones = jnp.ones((L,), jnp.float32)

    @pl.loop(0, n_pad // L)
    def _(i):
      hist[pl.ds(i * L, L)] = zvec

    @pl.loop(0, n_chunks, unroll=4)
    def _(j):
      for k in range(CHUNK // L):
        idx = row_v[j, pl.ds(k * L, L)]
        plsc.addupdate_scatter(hist, [idx], ones)

    for t in range(NS):
      pltpu.async_copy(hist.at[pl.ds(t * rows_per_tile, rows_per_tile)],
                       sumbuf.at[t, s], csem)
    for t in range(NS):
      pltpu.make_async_copy(hist.at[pl.ds(0, rows_per_tile)],
                            sumbuf.at[t, s], csem).wait()

    @pl.loop(0, CHUNK)
    def _(i):
      bufs[0, 0, i, :] = zvec

    @pl.loop(0, rows_per_tile // CHUNK)
    def _(jz):
      pltpu.sync_copy(bufs.at[0, 0],
                      acc.at[pl.ds(s * rows_per_tile + jz * CHUNK, CHUNK)])

    plsc.subcore_barrier()

    pltpu.sync_copy(xw1_hbm.at[pl.ds(s * rows_per_tile, rows_per_tile)],
                    xw_v)
    pltpu.sync_copy(sumbuf.at[s], comb)

    @pl.loop(0, nvec)
    def _(cidx):
      deg = comb[0, pl.ds(cidx * L, L)]
      for r in range(1, NS):
        deg = deg + comb[r, pl.ds(cidx * L, L)]
      dis_v[pl.ds(cidx * L, L)] = _rsqrt16(deg)

    @pl.loop(0, rows_per_tile, unroll=8)
    def _(r):
      dvec = plsc.load_gather(dis_v,
                              [jnp.zeros((L,), jnp.int32) + r])
      y_v[r, :] = xw_v[r, :] * dvec
      xw_v[r, :] = dvec

    pltpu.sync_copy(y_v, ytab.at[pl.ds(s * rows_per_tile,
                                       rows_per_tile)])
    pltpu.sync_copy(y_v, ytab_hbm.at[pl.ds(s * rows_per_tile,
                                           rows_per_tile)])
    pltpu.sync_copy(xw_v, dis_out.at[pl.ds(s * rows_per_tile,
                                           rows_per_tile)])

    plsc.subcore_barrier()

    _agg_phase((ytab, ytab_hbm), col_v, row_v, bufs, acc, gsems, ssems,
               n_chunks)

    plsc.subcore_barrier()

    pltpu.sync_copy(
        acc.at[pl.ds(s * rows_per_tile, rows_per_tile)],
        acc_out.at[pl.ds(s * rows_per_tile, rows_per_tile)])

  return pl.kernel(
      body,
      out_type=[
          jax.ShapeDtypeStruct((n_pad, F), jnp.float32),
          jax.ShapeDtypeStruct((n_pad, F), jnp.float32),
          jax.ShapeDtypeStruct((n_pad, F), jnp.float32),
      ],
      mesh=_sc_mesh(),
      compiler_params=pltpu.CompilerParams(use_tc_tiling_on_sc=False,
                                           needs_layout_passes=False),
      scratch_types=_AGG_SCRATCH(n_pad, n_chunks) + [
          pltpu.VMEM((n_pad,), jnp.float32),
          pltpu.VMEM_SHARED((NS, NS, n_pad // NS), jnp.float32),
          pltpu.VMEM((NS, n_pad // NS), jnp.float32),
          pltpu.VMEM((n_pad // NS, F), jnp.float32),
          pltpu.VMEM((n_pad // NS, F), jnp.float32),
          pltpu.VMEM((n_pad // NS,), jnp.float32),
          pltpu.SemaphoreType.DMA,
          pltpu.VMEM_SHARED((n_pad, F), jnp.float32),
      ],
  )


def _make_sc_agg(n_pad, n_chunks):
  rows_per_tile = n_pad // NS

  def body(y_hbm, cols_hbm, rows_hbm, out_hbm, col_v, row_v, bufs, acc,
           gsems, ssems, ytab):
    s = lax.axis_index("s")
    pltpu.sync_copy(y_hbm.at[pl.ds(s * rows_per_tile, rows_per_tile)],
                    ytab.at[pl.ds(s * rows_per_tile, rows_per_tile)])

    zvec = jnp.zeros((L,), jnp.float32)

    @pl.loop(0, CHUNK)
    def _(i):
      bufs[0, 0, i, :] = zvec

    @pl.loop(0, rows_per_tile // CHUNK)
    def _(jz):
      pltpu.sync_copy(bufs.at[0, 0],
                      acc.at[pl.ds(s * rows_per_tile + jz * CHUNK, CHUNK)])

    pltpu.sync_copy(cols_hbm.at[s], col_v)
    pltpu.sync_copy(rows_hbm.at[s], row_v)

    plsc.subcore_barrier()

    _agg_phase((ytab, y_hbm), col_v, row_v, bufs, acc, gsems, ssems,
               n_chunks)

    plsc.subcore_barrier()

    pltpu.sync_copy(
        acc.at[pl.ds(s * rows_per_tile, rows_per_tile)],
        out_hbm.at[pl.ds(s * rows_per_tile, rows_per_tile)])

  return pl.kernel(
      body,
      out_type=jax.ShapeDtypeStruct((n_pad, F), jnp.float32),
      mesh=_sc_mesh(),
      compiler_params=pltpu.CompilerParams(use_tc_tiling_on_sc=False,
                                           needs_layout_passes=False),
      scratch_types=_AGG_SCRATCH(n_pad, n_chunks) + [
          pltpu.VMEM_SHARED((n_pad, F), jnp.float32),
      ],
  )


_R = 2000


def _tca_body(x, w10, w11, xw0_o, xw1_o):
  xv = x[...]
  xw0_o[...] = jnp.dot(xv, w10[...], preferred_element_type=jnp.float32)
  xw1_o[...] = jnp.dot(xv, w11[...], preferred_element_type=jnp.float32)


def _tcb_body(a1, dis, xw0, b1, w20, w21, hw0_o, y2_o):
  h = jnp.maximum(xw0[...] - dis[...] * a1[...] + b1[...], 0.0)
  hw0_o[...] = jnp.dot(h, w20[...], preferred_element_type=jnp.float32)
  y2_o[...] = dis[...] * jnp.dot(h, w21[...],
                                 preferred_element_type=jnp.float32)


def _tcc_body(a2, hw0, dis, b2, out_o):
  z = hw0[...] - dis[...] * a2[...] + b2[...]
  m = jnp.max(z, axis=1, keepdims=True)
  e = jnp.exp(z - m)
  out_o[...] = (z - m) - jnp.log(jnp.sum(e, axis=1, keepdims=True))


def _row_spec(w):
  return pl.BlockSpec((_R, w), lambda i: (i, 0))


def _full_spec(shape):
  return pl.BlockSpec(shape, lambda i: tuple(0 for _ in shape))


def kernel(x, edge_index, W1, b1, W2, b2):
  n, f_in = x.shape
  e = edge_index.shape[1]
  hid = W1.shape[2]
  c_out = W2.shape[2]

  per_w_chunks = -(-e // (NS * CHUNK))
  n_chunks = -(-per_w_chunks // (2 * GRP)) * (2 * GRP)
  e_pad = NS * n_chunks * CHUNK
  n_pad = -(-n // (NS * CHUNK)) * (NS * CHUNK)

  row = edge_index[0]
  col = edge_index[1]
  pad = jnp.full((e_pad - e,), n, jnp.int32)
  rows3 = jnp.concatenate([row, pad]).reshape(NS, n_chunks, CHUNK)
  cols3 = jnp.concatenate([col, pad]).reshape(NS, n_chunks, CHUNK)

  grid = (n // _R,)
  xw0, xw1p = pl.pallas_call(
      _tca_body,
      grid=grid,
      in_specs=[
          _row_spec(f_in),
          _full_spec((f_in, hid)),
          _full_spec((f_in, hid)),
      ],
      out_specs=[_row_spec(hid), _row_spec(hid)],
      out_shape=[
          jax.ShapeDtypeStruct((n, hid), jnp.float32),
          jax.ShapeDtypeStruct((n_pad, hid), jnp.float32),
      ],
  )(x, W1[0], W1[1])

  acc1, disw, _ = _make_sc_layer1(n_pad, n_chunks)(cols3, rows3, xw1p)

  hw0, y2p = pl.pallas_call(
      _tcb_body,
      grid=grid,
      in_specs=[
          _row_spec(F),
          _row_spec(F),
          _row_spec(hid),
          _full_spec((1, hid)),
          _full_spec((hid, c_out)),
          _full_spec((hid, c_out)),
      ],
      out_specs=[_row_spec(c_out), _row_spec(c_out)],
      out_shape=[
          jax.ShapeDtypeStruct((n, c_out), jnp.float32),
          jax.ShapeDtypeStruct((n_pad, c_out), jnp.float32),
      ],
  )(acc1, disw, xw0, b1.reshape(1, hid), W2[0], W2[1])

  acc2 = _make_sc_agg(n_pad, n_chunks)(y2p, cols3, rows3)

  out = pl.pallas_call(
      _tcc_body,
      grid=grid,
      in_specs=[
          _row_spec(c_out),
          _row_spec(c_out),
          _row_spec(F),
          _full_spec((1, c_out)),
      ],
      out_specs=_row_spec(c_out),
      out_shape=jax.ShapeDtypeStruct((n, c_out), jnp.float32),
  )(acc2, hw0, disw, b2.reshape(1, c_out))

  return out


# speedup vs baseline: 1.1628x; 1.1628x over previous
import jax
import jax.numpy as jnp
from jax import lax
from jax.experimental import pallas as pl
from jax.experimental.pallas import tpu as pltpu
from jax.experimental.pallas import tpu_sc as plsc

NS = 16
L = 16
CHUNK = 128
GRP = 4
F = 16


def _sc_mesh():
  return plsc.VectorSubcoreMesh(
      core_axis_name="c", subcore_axis_name="s", num_cores=1,
      num_subcores=NS)


def _rsqrt16(x):
  xi = plsc.bitcast(x, jnp.int32)
  gi = jnp.int32(0x5F3759DF) - lax.shift_right_logical(xi, 1)
  g = plsc.bitcast(gi, jnp.float32)
  half = -0.5 * x
  for _ in range(3):
    g = g * (1.5 + half * g * g)
  return jnp.where(x > 0.0, g, 0.0)


def _agg_phase(y_hbm, col_v, row_v, bufs, acc, gsems, ssems, n_chunks):
  n_groups = n_chunks // GRP

  def gather(i):
    p = i % 2
    for b in range(GRP):
      j = i * GRP + b
      pltpu.async_copy(y_hbm.at[col_v.at[j]], bufs.at[p, b], gsems[p])

  gather(0)
  for i in range(n_groups):
    p = i % 2
    if i + 1 < n_groups:
      if i >= 1:
        for b in range(GRP):
          pltpu.make_async_copy(bufs.at[1 - p, b],
                                acc.at[row_v.at[b]], ssems[1 - p]).wait()
      gather(i + 1)
    for b in range(GRP):
      j = i * GRP + b
      pltpu.make_async_copy(y_hbm.at[col_v.at[j]], bufs.at[p, b],
                            gsems[p]).wait()
    for b in range(GRP):
      j = i * GRP + b
      pltpu.async_copy(bufs.at[p, b], acc.at[row_v.at[j]], ssems[p],
                       add=True)
  for i in (n_groups - 2, n_groups - 1):
    p = i % 2
    for b in range(GRP):
      pltpu.make_async_copy(bufs.at[p, b], acc.at[row_v.at[b]],
                            ssems[p]).wait()


_AGG_SCRATCH = lambda n_pad, n_chunks: [
    pltpu.VMEM((n_chunks, CHUNK), jnp.int32),
    pltpu.VMEM((n_chunks, CHUNK), jnp.int32),
    pltpu.VMEM((2, GRP, CHUNK, F), jnp.float32),
    pltpu.VMEM_SHARED((n_pad, F), jnp.float32),
    [pltpu.SemaphoreType.DMA, pltpu.SemaphoreType.DMA],
    [pltpu.SemaphoreType.DMA, pltpu.SemaphoreType.DMA],
]


def _make_sc_layer1(n_pad, n_chunks):
  rows_per_tile = n_pad // NS
  nvec = rows_per_tile // L

  def body(cols_hbm, rows_hbm, xw1_hbm, acc_out, dis_out,
           col_v, row_v, bufs, acc, gsems, ssems,
           hist, sumbuf, comb, xw_v, y_v, dis_v, csem, ytab):
    s = lax.axis_index("s")

    pltpu.sync_copy(cols_hbm.at[s], col_v)
    pltpu.sync_copy(rows_hbm.at[s], row_v)

    zvec = jnp.zeros((L,), jnp.float32)
    ones = jnp.ones((L,), jnp.float32)

    @pl.loop(0, n_pad // L)
    def _(i):
      hist[pl.ds(i * L, L)] = zvec

    @pl.loop(0, n_chunks, unroll=4)
    def _(j):
      for k in range(CHUNK // L):
        idx = row_v[j, pl.ds(k * L, L)]
        plsc.addupdate_scatter(hist, [idx], ones)

    for t in range(NS):
      pltpu.async_copy(hist.at[pl.ds(t * rows_per_tile, rows_per_tile)],
                       sumbuf.at[t, s], csem)
    for t in range(NS):
      pltpu.make_async_copy(hist.at[pl.ds(0, rows_per_tile)],
                            sumbuf.at[t, s], csem).wait()

    @pl.loop(0, CHUNK)
    def _(i):
      bufs[0, 0, i, :] = zvec

    @pl.loop(0, rows_per_tile // CHUNK)
    def _(jz):
      pltpu.sync_copy(bufs.at[0, 0],
                      acc.at[pl.ds(s * rows_per_tile + jz * CHUNK, CHUNK)])

    plsc.subcore_barrier()

    pltpu.sync_copy(xw1_hbm.at[pl.ds(s * rows_per_tile, rows_per_tile)],
                    xw_v)
    pltpu.sync_copy(sumbuf.at[s], comb)

    @pl.loop(0, nvec)
    def _(cidx):
      deg = comb[0, pl.ds(cidx * L, L)]
      for r in range(1, NS):
        deg = deg + comb[r, pl.ds(cidx * L, L)]
      dis_v[pl.ds(cidx * L, L)] = _rsqrt16(deg)

    @pl.loop(0, rows_per_tile, unroll=8)
    def _(r):
      dvec = plsc.load_gather(dis_v,
                              [jnp.zeros((L,), jnp.int32) + r])
      y_v[r, :] = xw_v[r, :] * dvec
      xw_v[r, :] = dvec

    pltpu.sync_copy(y_v, ytab.at[pl.ds(s * rows_per_tile,
                                       rows_per_tile)])
    pltpu.sync_copy(xw_v, dis_out.at[pl.ds(s * rows_per_tile,
                                           rows_per_tile)])

    plsc.subcore_barrier()

    _agg_phase(ytab, col_v, row_v, bufs, acc, gsems, ssems, n_chunks)

    plsc.subcore_barrier()

    pltpu.sync_copy(
        acc.at[pl.ds(s * rows_per_tile, rows_per_tile)],
        acc_out.at[pl.ds(s * rows_per_tile, rows_per_tile)])

  return pl.kernel(
      body,
      out_type=[
          jax.ShapeDtypeStruct((n_pad, F), jnp.float32),
          jax.ShapeDtypeStruct((n_pad, F), jnp.float32),
      ],
      mesh=_sc_mesh(),
      compiler_params=pltpu.CompilerParams(use_tc_tiling_on_sc=False,
                                           needs_layout_passes=False),
      scratch_types=_AGG_SCRATCH(n_pad, n_chunks) + [
          pltpu.VMEM((n_pad,), jnp.float32),
          pltpu.VMEM_SHARED((NS, NS, n_pad // NS), jnp.float32),
          pltpu.VMEM((NS, n_pad // NS), jnp.float32),
          pltpu.VMEM((n_pad // NS, F), jnp.float32),
          pltpu.VMEM((n_pad // NS, F), jnp.float32),
          pltpu.VMEM((n_pad // NS,), jnp.float32),
          pltpu.SemaphoreType.DMA,
          pltpu.VMEM_SHARED((n_pad, F), jnp.float32),
      ],
  )


def _make_sc_agg(n_pad, n_chunks):
  rows_per_tile = n_pad // NS

  def body(y_hbm, cols_hbm, rows_hbm, out_hbm, col_v, row_v, bufs, acc,
           gsems, ssems, ytab):
    s = lax.axis_index("s")
    pltpu.sync_copy(y_hbm.at[pl.ds(s * rows_per_tile, rows_per_tile)],
                    ytab.at[pl.ds(s * rows_per_tile, rows_per_tile)])

    zvec = jnp.zeros((L,), jnp.float32)

    @pl.loop(0, CHUNK)
    def _(i):
      bufs[0, 0, i, :] = zvec

    @pl.loop(0, rows_per_tile // CHUNK)
    def _(jz):
      pltpu.sync_copy(bufs.at[0, 0],
                      acc.at[pl.ds(s * rows_per_tile + jz * CHUNK, CHUNK)])

    pltpu.sync_copy(cols_hbm.at[s], col_v)
    pltpu.sync_copy(rows_hbm.at[s], row_v)

    plsc.subcore_barrier()

    _agg_phase(ytab, col_v, row_v, bufs, acc, gsems, ssems, n_chunks)

    plsc.subcore_barrier()

    pltpu.sync_copy(
        acc.at[pl.ds(s * rows_per_tile, rows_per_tile)],
        out_hbm.at[pl.ds(s * rows_per_tile, rows_per_tile)])

  return pl.kernel(
      body,
      out_type=jax.ShapeDtypeStruct((n_pad, F), jnp.float32),
      mesh=_sc_mesh(),
      compiler_params=pltpu.CompilerParams(use_tc_tiling_on_sc=False,
                                           needs_layout_passes=False),
      scratch_types=_AGG_SCRATCH(n_pad, n_chunks) + [
          pltpu.VMEM_SHARED((n_pad, F), jnp.float32),
      ],
  )


_R = 2000


def _tca_body(x, w10, w11, xw0_o, xw1_o):
  xv = x[...]
  xw0_o[...] = jnp.dot(xv, w10[...], preferred_element_type=jnp.float32)
  xw1_o[...] = jnp.dot(xv, w11[...], preferred_element_type=jnp.float32)


def _tcb_body(a1, dis, xw0, b1, w20, w21, hw0_o, y2_o):
  h = jnp.maximum(xw0[...] - dis[...] * a1[...] + b1[...], 0.0)
  hw0_o[...] = jnp.dot(h, w20[...], preferred_element_type=jnp.float32)
  y2_o[...] = dis[...] * jnp.dot(h, w21[...],
                                 preferred_element_type=jnp.float32)


def _tcc_body(a2, hw0, dis, b2, out_o):
  z = hw0[...] - dis[...] * a2[...] + b2[...]
  m = jnp.max(z, axis=1, keepdims=True)
  e = jnp.exp(z - m)
  out_o[...] = (z - m) - jnp.log(jnp.sum(e, axis=1, keepdims=True))


def _row_spec(w):
  return pl.BlockSpec((_R, w), lambda i: (i, 0))


def _full_spec(shape):
  return pl.BlockSpec(shape, lambda i: tuple(0 for _ in shape))


def kernel(x, edge_index, W1, b1, W2, b2):
  n, f_in = x.shape
  e = edge_index.shape[1]
  hid = W1.shape[2]
  c_out = W2.shape[2]

  per_w_chunks = -(-e // (NS * CHUNK))
  n_chunks = -(-per_w_chunks // (2 * GRP)) * (2 * GRP)
  e_pad = NS * n_chunks * CHUNK
  n_pad = -(-n // (NS * CHUNK)) * (NS * CHUNK)

  row = edge_index[0]
  col = edge_index[1]
  pad = jnp.full((e_pad - e,), n, jnp.int32)
  rows3 = jnp.concatenate([row, pad]).reshape(NS, n_chunks, CHUNK)
  cols3 = jnp.concatenate([col, pad]).reshape(NS, n_chunks, CHUNK)

  grid = (n // _R,)
  xw0, xw1p = pl.pallas_call(
      _tca_body,
      grid=grid,
      in_specs=[
          _row_spec(f_in),
          _full_spec((f_in, hid)),
          _full_spec((f_in, hid)),
      ],
      out_specs=[_row_spec(hid), _row_spec(hid)],
      out_shape=[
          jax.ShapeDtypeStruct((n, hid), jnp.float32),
          jax.ShapeDtypeStruct((n_pad, hid), jnp.float32),
      ],
  )(x, W1[0], W1[1])

  acc1, disw = _make_sc_layer1(n_pad, n_chunks)(cols3, rows3, xw1p)

  hw0, y2p = pl.pallas_call(
      _tcb_body,
      grid=grid,
      in_specs=[
          _row_spec(F),
          _row_spec(F),
          _row_spec(hid),
          _full_spec((1, hid)),
          _full_spec((hid, c_out)),
          _full_spec((hid, c_out)),
      ],
      out_specs=[_row_spec(c_out), _row_spec(c_out)],
      out_shape=[
          jax.ShapeDtypeStruct((n, c_out), jnp.float32),
          jax.ShapeDtypeStruct((n_pad, c_out), jnp.float32),
      ],
  )(acc1, disw, xw0, b1.reshape(1, hid), W2[0], W2[1])

  acc2 = _make_sc_agg(n_pad, n_chunks)(y2p, cols3, rows3)

  out = pl.pallas_call(
      _tcc_body,
      grid=grid,
      in_specs=[
          _row_spec(c_out),
          _row_spec(c_out),
          _row_spec(F),
          _full_spec((1, c_out)),
      ],
      out_specs=_row_spec(c_out),
      out_shape=jax.ShapeDtypeStruct((n, c_out), jnp.float32),
  )(acc2, hw0, disw, b2.reshape(1, c_out))

  return out
